# baseline (speedup 1.0000x reference)
import jax
import jax.numpy as jnp
from jax.experimental import pallas as pl
from jax.experimental.pallas import tpu as pltpu

_BSZ, _SEQ, _HID = 4, 8192, 768
_E = 8
_ALPHA = 0.1
_BLK = 4096
_NTOK = _BSZ * _SEQ
_NBLK = _NTOK // _BLK
_BLK_PER_BATCH = _SEQ // _BLK


def _router_body(x_ref, wt_ref, idx_ref, wgt_ref, aux_ref, cnt_ref, ssum_ref):
    i = pl.program_id(0)
    b = i // _BLK_PER_BATCH

    @pl.when(i == 0)
    def _init():
        cnt_ref[...] = jnp.zeros((_BSZ * _E, 1), jnp.float32)
        ssum_ref[...] = jnp.zeros((_BSZ * _E, 1), jnp.float32)

    x = x_ref[...]
    wt = wt_ref[...]
    logits = jax.lax.dot_general(
        x, wt, (((1,), (0,)), ((), ())), preferred_element_type=jnp.float32
    )
    lt = logits.T

    m = jnp.max(lt, axis=0, keepdims=True)
    ex = jnp.exp(lt - m)
    s = jnp.sum(ex, axis=0, keepdims=True)
    scores = ex / s

    row = jax.lax.broadcasted_iota(jnp.int32, (_E, _BLK), 0)
    w1 = jnp.max(scores, axis=0, keepdims=True)
    idx1 = jnp.argmax(scores, axis=0).reshape(1, _BLK)
    masked = jnp.where(row == idx1, -jnp.inf, scores)
    w2 = jnp.max(masked, axis=0, keepdims=True)
    idx2 = jnp.argmax(masked, axis=0).reshape(1, _BLK)

    inv_denom = 1.0 / (w1 + w2 + 1e-20)
    wgt_ref[...] = jnp.concatenate([w1 * inv_denom, w2 * inv_denom], axis=0)
    idx_ref[...] = jnp.concatenate([idx1, idx2], axis=0)

    onehot = jnp.where(row == idx1, 1.0, 0.0) + jnp.where(row == idx2, 1.0, 0.0)
    blk_cnt = jnp.sum(onehot, axis=1, keepdims=True)
    blk_ssum = jnp.sum(scores, axis=1, keepdims=True)
    cnt_ref[pl.ds(b * _E, _E), :] = cnt_ref[pl.ds(b * _E, _E), :] + blk_cnt
    ssum_ref[pl.ds(b * _E, _E), :] = ssum_ref[pl.ds(b * _E, _E), :] + blk_ssum

    @pl.when(i == _NBLK - 1)
    def _fin():
        ce = cnt_ref[...] * (_E / (_SEQ * 2.0))
        smean = ssum_ref[...] / _SEQ
        aux_ref[0, 0] = jnp.sum(ce * smean) / _BSZ * _ALPHA


def kernel(x, weight):
    xf = x.reshape(_NTOK, _HID)
    wt = weight.T
    idx_t, wgt_t, aux = pl.pallas_call(
        _router_body,
        grid=(_NBLK,),
        in_specs=[
            pl.BlockSpec((_BLK, _HID), lambda i: (i, 0)),
            pl.BlockSpec((_HID, _E), lambda i: (0, 0)),
        ],
        out_specs=[
            pl.BlockSpec((2, _BLK), lambda i: (0, i)),
            pl.BlockSpec((2, _BLK), lambda i: (0, i)),
            pl.BlockSpec(memory_space=pltpu.SMEM),
        ],
        out_shape=[
            jax.ShapeDtypeStruct((2, _NTOK), jnp.int32),
            jax.ShapeDtypeStruct((2, _NTOK), jnp.float32),
            jax.ShapeDtypeStruct((1, 1), jnp.float32),
        ],
        scratch_shapes=[
            pltpu.VMEM((_BSZ * _E, 1), jnp.float32),
            pltpu.VMEM((_BSZ * _E, 1), jnp.float32),
        ],
        compiler_params=pltpu.CompilerParams(
            dimension_semantics=("arbitrary",),
        ),
    )(xf, wt)
    return idx_t, wgt_t, aux[0, 0]


# speedup vs baseline: 1.0111x; 1.0111x over previous
import jax
import jax.numpy as jnp
from jax.experimental import pallas as pl
from jax.experimental.pallas import tpu as pltpu

_BSZ, _SEQ, _HID = 4, 8192, 768
_E = 8
_ALPHA = 0.1
_BLK = 4096
_NTOK = _BSZ * _SEQ
_NBLK = _NTOK // _BLK
_BLK_PER_BATCH = _SEQ // _BLK


def _router_body(x_ref, wt_ref, idx_ref, wgt_ref, aux_ref, cnt_ref, ssum_ref):
    i = pl.program_id(0)
    b = i // _BLK_PER_BATCH

    @pl.when(i == 0)
    def _init():
        cnt_ref[...] = jnp.zeros((_BSZ * _E, 1), jnp.float32)
        ssum_ref[...] = jnp.zeros((_BSZ * _E, 1), jnp.float32)

    x = x_ref[...]
    wt = wt_ref[...]
    logits = jax.lax.dot_general(
        x, wt, (((1,), (0,)), ((), ())), preferred_element_type=jnp.float32
    )
    lt = logits.T

    m = jnp.max(lt, axis=0, keepdims=True)
    ex = jnp.exp(lt - m)
    s = jnp.sum(ex, axis=0, keepdims=True)
    scores = ex / s

    row = jax.lax.broadcasted_iota(jnp.int32, (_E, _BLK), 0)
    w1 = jnp.max(scores, axis=0, keepdims=True)
    idx1 = jnp.argmax(scores, axis=0).reshape(1, _BLK)
    masked = jnp.where(row == idx1, -jnp.inf, scores)
    w2 = jnp.max(masked, axis=0, keepdims=True)
    idx2 = jnp.argmax(masked, axis=0).reshape(1, _BLK)

    inv_denom = 1.0 / (w1 + w2 + 1e-20)
    wgt_ref[...] = jnp.concatenate([w1 * inv_denom, w2 * inv_denom], axis=0)
    idx_ref[...] = jnp.concatenate([idx1, idx2], axis=0)

    onehot = jnp.where(row == idx1, 1.0, 0.0) + jnp.where(row == idx2, 1.0, 0.0)
    blk_cnt = jnp.sum(onehot, axis=1, keepdims=True)
    blk_ssum = jnp.sum(scores, axis=1, keepdims=True)
    cnt_ref[pl.ds(b * _E, _E), :] = cnt_ref[pl.ds(b * _E, _E), :] + blk_cnt
    ssum_ref[pl.ds(b * _E, _E), :] = ssum_ref[pl.ds(b * _E, _E), :] + blk_ssum

    @pl.when(i == _NBLK - 1)
    def _fin():
        ce = cnt_ref[...] * (_E / (_SEQ * 2.0))
        smean = ssum_ref[...] / _SEQ
        aux_ref[0, 0] = jnp.sum(ce * smean) / _BSZ * _ALPHA


def kernel(x, weight):
    xf = x.reshape(_NTOK, _HID)
    wt = weight.T
    idx_t, wgt_t, aux = pl.pallas_call(
        _router_body,
        grid=(_NBLK,),
        in_specs=[
            pl.BlockSpec((_BLK, _HID), lambda i: (i, 0)),
            pl.BlockSpec((_HID, _E), lambda i: (0, 0)),
        ],
        out_specs=[
            pl.BlockSpec((2, _BLK), lambda i: (0, i)),
            pl.BlockSpec((2, _BLK), lambda i: (0, i)),
            pl.BlockSpec(memory_space=pltpu.SMEM),
        ],
        out_shape=[
            jax.ShapeDtypeStruct((2, _NTOK), jnp.int32),
            jax.ShapeDtypeStruct((2, _NTOK), jnp.float32),
            jax.ShapeDtypeStruct((1, 1), jnp.float32),
        ],
        scratch_shapes=[
            pltpu.VMEM((_BSZ * _E, 1), jnp.float32),
            pltpu.VMEM((_BSZ * _E, 1), jnp.float32),
        ],
        compiler_params=pltpu.CompilerParams(
            dimension_semantics=("arbitrary",),
        ),
    )(xf, wt)
    return idx_t.T, wgt_t.T, aux[0, 0]
